# baseline (device time: 15343 ns/iter reference)
import jax
import jax.numpy as jnp
from jax import lax
from jax.experimental import pallas as pl
from jax.experimental.pallas import tpu as pltpu

M_PER = 1024
N_COLS = 512
CHUNKS = 4
ROWS_C = M_PER // CHUNKS


def kernel(x, dest):
    dest2d = dest.astype(jnp.int32).reshape(1, M_PER)

    def body(x_ref, dest_ref, out_ref,
             perm_buf, x_peer, cnt_send, cnt_peer, send_sems, recv_sems):
        my_x = lax.axis_index("x")
        my_y = lax.axis_index("y")
        partner = (my_x, 1 - my_y)

        barrier_sem = pltpu.get_barrier_semaphore()
        pl.semaphore_signal(
            barrier_sem, inc=1,
            device_id=partner, device_id_type=pl.DeviceIdType.MESH,
        )

        iota_c = lax.broadcasted_iota(jnp.int32, (1, M_PER), 1)
        mask_m = dest_ref[:, :] == my_y
        cum = mask_m.astype(jnp.int32)
        s = 1
        while s < M_PER:
            cum = cum + jnp.where(iota_c >= s, pltpu.roll(cum, s, 1), 0)
            s *= 2
        cum_m = cum
        tot_m = jnp.max(cum_m)
        c_s = M_PER - tot_m
        slot = jnp.where(mask_m, c_s + cum_m - 1, iota_c - cum_m)
        xv = x_ref[:, :].astype(jnp.bfloat16)

        pl.semaphore_wait(barrier_sem, 1)

        cnt_send[:, :] = jnp.zeros((1, 128), jnp.int32) + c_s
        rdma_a = pltpu.make_async_remote_copy(
            src_ref=cnt_send, dst_ref=cnt_peer,
            send_sem=send_sems.at[0], recv_sem=recv_sems.at[0],
            device_id=partner, device_id_type=pl.DeviceIdType.MESH,
        )
        rdma_a.start()

        rdma_x = []
        for q in range(CHUNKS):
            sl = pl.ds(q * ROWS_C, ROWS_C)
            r = pltpu.make_async_remote_copy(
                src_ref=perm_buf.at[sl, :],
                dst_ref=x_peer.at[sl, :],
                send_sem=send_sems.at[1 + q], recv_sem=recv_sems.at[1 + q],
                device_id=partner, device_id_type=pl.DeviceIdType.MESH,
            )
            rdma_x.append(r)

            iota_b = (lax.broadcasted_iota(jnp.int32, (ROWS_C, M_PER), 0)
                      + q * ROWS_C)
            Pb = (iota_b == slot).astype(jnp.float32).astype(jnp.bfloat16)
            perm_buf[sl, :] = jnp.dot(
                Pb, xv, preferred_element_type=jnp.float32,
            ).astype(jnp.bfloat16)

            @pl.when(q * ROWS_C < c_s)
            def _(r=r):
                r.start()

        rdma_a.wait()
        c_r = jnp.max(cnt_peer[:, :])
        off_m = jnp.where(my_y == 0, 0, c_r)
        off_p = jnp.where(my_y == 0, tot_m, 0)

        rolled_m = pltpu.roll(perm_buf[:, :], (off_m - c_s) % M_PER, 0)
        row_i = lax.broadcasted_iota(jnp.int32, (M_PER, 1), 0)
        in_peer = (row_i >= off_p) & (row_i < off_p + c_r)

        for q in range(CHUNKS):
            @pl.when(q * ROWS_C < c_r)
            def _(q=q):
                rdma_x[q].wait_recv()
        rolled_p = pltpu.roll(x_peer[:, :], off_p, 0)
        out_ref[:, :] = jnp.where(
            in_peer, rolled_p, rolled_m).astype(jnp.float32)

        for q in range(CHUNKS):
            @pl.when(q * ROWS_C < c_s)
            def _(q=q):
                rdma_x[q].wait_send()

    return pl.pallas_call(
        body,
        out_shape=jax.ShapeDtypeStruct((M_PER, N_COLS), jnp.float32),
        in_specs=[
            pl.BlockSpec(memory_space=pltpu.VMEM),
            pl.BlockSpec(memory_space=pltpu.VMEM),
        ],
        out_specs=pl.BlockSpec(memory_space=pltpu.VMEM),
        scratch_shapes=[
            pltpu.VMEM((M_PER, N_COLS), jnp.bfloat16),
            pltpu.VMEM((M_PER, N_COLS), jnp.bfloat16),
            pltpu.VMEM((1, 128), jnp.int32),
            pltpu.VMEM((1, 128), jnp.int32),
            pltpu.SemaphoreType.DMA((1 + CHUNKS,)),
            pltpu.SemaphoreType.DMA((1 + CHUNKS,)),
        ],
        compiler_params=pltpu.CompilerParams(collective_id=0),
    )(x, dest2d)


# device time: 14754 ns/iter; 1.0399x vs baseline; 1.0399x over previous
import jax
import jax.numpy as jnp
from jax import lax
from jax.experimental import pallas as pl
from jax.experimental.pallas import tpu as pltpu

M_PER = 1024
N_COLS = 512
CHUNKS = 4
ROWS_C = M_PER // CHUNKS


def kernel(x, dest):
    dest2d = dest.astype(jnp.int32).reshape(1, M_PER)

    def body(x_ref, dest_ref, out_ref,
             send_buf, x_peer, aux_send, aux_peer, send_sems, recv_sems):
        my_x = lax.axis_index("x")
        my_y = lax.axis_index("y")
        partner = (my_x, 1 - my_y)

        barrier_sem = pltpu.get_barrier_semaphore()
        pl.semaphore_signal(
            barrier_sem, inc=1,
            device_id=partner, device_id_type=pl.DeviceIdType.MESH,
        )
        pl.semaphore_wait(barrier_sem, 1)

        iota_p = lax.broadcasted_iota(jnp.int32, (M_PER, M_PER), 0)
        iota_l = lax.broadcasted_iota(jnp.int32, (M_PER, M_PER), 1)
        tri = (iota_p <= iota_l).astype(jnp.float32)
        iota_c = lax.broadcasted_iota(jnp.int32, (1, M_PER), 1)

        mask_m = dest_ref[:, :] == my_y
        cum_m = jnp.dot(mask_m.astype(jnp.float32), tri,
                        preferred_element_type=jnp.float32).astype(jnp.int32)
        tot_m = jnp.max(cum_m)
        c_s = M_PER - tot_m
        cum_pt = (iota_c + 1) - cum_m

        pos_send = cum_pt - 1
        xv = x_ref[:, :].astype(jnp.bfloat16)

        aux_send[:, :] = jnp.zeros((1, 128), jnp.int32) + c_s
        rdma_a = pltpu.make_async_remote_copy(
            src_ref=aux_send, dst_ref=aux_peer,
            send_sem=send_sems.at[0], recv_sem=recv_sems.at[0],
            device_id=partner, device_id_type=pl.DeviceIdType.MESH,
        )
        rdma_a.start()

        rdma_x = []
        for q in range(CHUNKS):
            sl = pl.ds(q * ROWS_C, ROWS_C)
            r = pltpu.make_async_remote_copy(
                src_ref=send_buf.at[sl, :],
                dst_ref=x_peer.at[sl, :],
                send_sem=send_sems.at[1 + q], recv_sem=recv_sems.at[1 + q],
                device_id=partner, device_id_type=pl.DeviceIdType.MESH,
            )
            rdma_x.append(r)

            @pl.when(q * ROWS_C < c_s)
            def _(q=q, r=r, sl=sl):
                iota_b = (lax.broadcasted_iota(jnp.int32, (ROWS_C, M_PER), 0)
                          + q * ROWS_C)
                Pb = (((iota_b == pos_send) & (~mask_m))
                      .astype(jnp.float32).astype(jnp.bfloat16))
                send_buf[sl, :] = jnp.dot(
                    Pb, xv, preferred_element_type=jnp.float32,
                ).astype(jnp.bfloat16)
                r.start()

        rdma_a.wait()
        c_r = jnp.max(aux_peer[:, :])
        off_m = jnp.where(my_y == 0, 0, c_r)
        off_p = jnp.where(my_y == 0, tot_m, 0)

        P_m = (((iota_p == (cum_m - 1 + off_m)) & mask_m)
               .astype(jnp.float32).astype(jnp.bfloat16))
        acc = jnp.dot(P_m, xv, preferred_element_type=jnp.float32)

        for q in range(CHUNKS):
            @pl.when(q * ROWS_C < c_r)
            def _(q=q):
                rdma_x[q].wait_recv()
        rolled = pltpu.roll(x_peer[:, :], off_p, 0).astype(jnp.float32)
        row_i = lax.broadcasted_iota(jnp.int32, (M_PER, 1), 0)
        in_peer = (row_i >= off_p) & (row_i < off_p + c_r)
        out_ref[:, :] = jnp.where(in_peer, rolled, acc)

        for q in range(CHUNKS):
            @pl.when(q * ROWS_C < c_s)
            def _(q=q):
                rdma_x[q].wait_send()

    return pl.pallas_call(
        body,
        out_shape=jax.ShapeDtypeStruct((M_PER, N_COLS), jnp.float32),
        in_specs=[
            pl.BlockSpec(memory_space=pltpu.VMEM),
            pl.BlockSpec(memory_space=pltpu.VMEM),
        ],
        out_specs=pl.BlockSpec(memory_space=pltpu.VMEM),
        scratch_shapes=[
            pltpu.VMEM((M_PER, N_COLS), jnp.bfloat16),
            pltpu.VMEM((M_PER, N_COLS), jnp.bfloat16),
            pltpu.VMEM((1, 128), jnp.int32),
            pltpu.VMEM((1, 128), jnp.int32),
            pltpu.SemaphoreType.DMA((1 + CHUNKS,)),
            pltpu.SemaphoreType.DMA((1 + CHUNKS,)),
        ],
        compiler_params=pltpu.CompilerParams(collective_id=0),
    )(x, dest2d)
